# Initial kernel scaffold; baseline (speedup 1.0000x reference)
import jax
import jax.numpy as jnp
from jax.experimental import pallas as pl


def kernel(encoded_tokens, pos_table):
    raise NotImplementedError("write your pallas kernel here")


# speedup vs baseline: 1.4885x; 1.4885x over previous
import jax
import jax.numpy as jnp
from jax.experimental import pallas as pl
from jax.experimental.pallas import tpu as pltpu

_BATCH = 4
_NUM_TOKENS = 8192
_EMBED_DIM = 1024
_T_BLK = 512


def _add_body(tok_ref, tab_ref, out_ref):
    out_ref[...] = tok_ref[...] + tab_ref[...]


def kernel(encoded_tokens, pos_table):
    return pl.pallas_call(
        _add_body,
        grid=(_NUM_TOKENS // _T_BLK, _BATCH),
        in_specs=[
            pl.BlockSpec((1, _T_BLK, _EMBED_DIM), lambda i, b: (b, i, 0)),
            pl.BlockSpec((_T_BLK, _EMBED_DIM), lambda i, b: (i, 0)),
        ],
        out_specs=pl.BlockSpec((1, _T_BLK, _EMBED_DIM), lambda i, b: (b, i, 0)),
        out_shape=jax.ShapeDtypeStruct(
            (_BATCH, _NUM_TOKENS, _EMBED_DIM), jnp.float32
        ),
        compiler_params=pltpu.CompilerParams(
            dimension_semantics=("arbitrary", "arbitrary"),
        ),
    )(encoded_tokens, pos_table)
